# Initial kernel scaffold; baseline (speedup 1.0000x reference)
import jax
import jax.numpy as jnp
from jax.experimental import pallas as pl


def kernel(x, edge_index, edge_attr, W1, root1, b1, W2, root2, b2):
    raise NotImplementedError("write your pallas kernel here")


# speedup vs baseline: 8.9176x; 8.9176x over previous
import functools

import jax
import jax.numpy as jnp
from jax import lax
from jax.experimental import pallas as pl
from jax.experimental.pallas import tpu as pltpu
from jax.experimental.pallas import tpu_sc as plsc

SUB = 80
CHUNK_ROWS = 16
NWORKERS = 32
LANES = 16


def _proj_body(x_ref, wud_ref, wr_ref, b_ref, t_ref, r_ref):
    x = x_ref[...]
    t_ref[...] = jnp.dot(x, wud_ref[...], preferred_element_type=jnp.float32)
    r_ref[...] = jnp.dot(x, wr_ref[...], preferred_element_type=jnp.float32) + b_ref[...]


def _mid_body(aggp_ref, r1_ref, wud_ref, wr_ref, b_ref, t_ref, r_ref):
    a = aggp_ref[...]
    s = a[0] + a[1]
    mean = s[:, :16] / jnp.maximum(s[:, 16:17], 1.0)
    t = mean + r1_ref[...]
    h = jnp.where(t > 0.0, t, jnp.exp(jnp.minimum(t, 0.0)) - 1.0)
    t_ref[...] = jnp.dot(h, wud_ref[...], preferred_element_type=jnp.float32)
    r_ref[...] = jnp.dot(h, wr_ref[...], preferred_element_type=jnp.float32) + b_ref[...]


def _out_body(aggp_ref, cntp_ref, r2_ref, o_ref):
    a = aggp_ref[...]
    c = cntp_ref[...]
    y = (a[0] + a[1]) / jnp.maximum(c[0] + c[1], 1.0) + r2_ref[...]
    m = jnp.max(y, axis=1, keepdims=True)
    e = y - m
    lse = jnp.log(jnp.sum(jnp.exp(e), axis=1, keepdims=True))
    o_ref[...] = e - lse


def _tc_proj(x, wud, wr, brow, bn):
    n, fin = x.shape
    fo = wud.shape[1]
    fr = wr.shape[1]
    grid = n // bn
    return pl.pallas_call(
        _proj_body,
        grid=(grid,),
        in_specs=[
            pl.BlockSpec((bn, fin), lambda i: (i, 0)),
            pl.BlockSpec((fin, fo), lambda i: (0, 0)),
            pl.BlockSpec((fin, fr), lambda i: (0, 0)),
            pl.BlockSpec((1, fr), lambda i: (0, 0)),
        ],
        out_specs=[
            pl.BlockSpec((bn, fo), lambda i: (i, 0)),
            pl.BlockSpec((bn, fr), lambda i: (i, 0)),
        ],
        out_shape=[
            jax.ShapeDtypeStruct((n, fo), jnp.float32),
            jax.ShapeDtypeStruct((n, fr), jnp.float32),
        ],
    )(x, wud, wr, brow)


def _tc_mid(aggp, r1, wud, wr, brow, bn):
    n, f = r1.shape
    fa = aggp.shape[2]
    fo = wud.shape[1]
    fr = wr.shape[1]
    grid = n // bn
    return pl.pallas_call(
        _mid_body,
        grid=(grid,),
        in_specs=[
            pl.BlockSpec((2, bn, fa), lambda i: (0, i, 0)),
            pl.BlockSpec((bn, f), lambda i: (i, 0)),
            pl.BlockSpec((f, fo), lambda i: (0, 0)),
            pl.BlockSpec((f, fr), lambda i: (0, 0)),
            pl.BlockSpec((1, fr), lambda i: (0, 0)),
        ],
        out_specs=[
            pl.BlockSpec((bn, fo), lambda i: (i, 0)),
            pl.BlockSpec((bn, fr), lambda i: (i, 0)),
        ],
        out_shape=[
            jax.ShapeDtypeStruct((n, fo), jnp.float32),
            jax.ShapeDtypeStruct((n, fr), jnp.float32),
        ],
    )(aggp, r1, wud, wr, brow)


def _tc_out(aggp, cntp, r2, bn):
    n, f = r2.shape
    grid = n // bn
    return pl.pallas_call(
        _out_body,
        grid=(grid,),
        in_specs=[
            pl.BlockSpec((2, bn, f), lambda i: (0, i, 0)),
            pl.BlockSpec((2, bn, 1), lambda i: (0, i, 0)),
            pl.BlockSpec((bn, f), lambda i: (i, 0)),
        ],
        out_specs=pl.BlockSpec((bn, f), lambda i: (i, 0)),
        out_shape=jax.ShapeDtypeStruct((n, f), jnp.float32),
    )(aggp, cntp, r2)


def _make_edge_kernel(n_nodes, n_idx_rows, feat, with_cnt):
    rows_per_tile = n_idx_rows // NWORKERS
    n_chunks = rows_per_tile // CHUNK_ROWS
    c_edges = CHUNK_ROWS * SUB
    n_read = n_nodes // LANES
    fw = feat + 16 if with_cnt else feat
    mesh = plsc.VectorSubcoreMesh(core_axis_name="c", subcore_axis_name="s")

    out_type = jax.ShapeDtypeStruct((2 * n_nodes, fw), jnp.float32)
    scratch = [
        pltpu.VMEM((CHUNK_ROWS, SUB), jnp.int32),
        pltpu.VMEM((CHUNK_ROWS, SUB), jnp.int32),
        pltpu.VMEM((c_edges,), jnp.float32),
        pltpu.VMEM((c_edges, 2 * feat), jnp.float32),
        pltpu.VMEM((c_edges, fw), jnp.float32),
        pltpu.VMEM_SHARED((n_nodes, fw), jnp.float32),
        pltpu.SemaphoreType.DMA,
    ]

    @functools.partial(pl.kernel, mesh=mesh, out_type=out_type,
                       scratch_types=scratch,
                       compiler_params=pltpu.CompilerParams(use_tc_tiling_on_sc=False))
    def edge_kernel(*refs):
        if with_cnt:
            (t_hbm, src_hbm, dst_hbm, p_hbm, zf_hbm, pat_hbm,
             agg_out,
             src_v, dst_v, p_v, rows_v, msg_v, agg_sh, sem) = refs
        else:
            (t_hbm, src_hbm, dst_hbm, p_hbm, zf_hbm,
             agg_out,
             src_v, dst_v, p_v, rows_v, msg_v, agg_sh, sem) = refs
        cid = lax.axis_index("c")
        sid = lax.axis_index("s")
        wid = cid * 16 + sid

        @pl.when(sid == 0)
        def _():
            pltpu.sync_copy(zf_hbm, agg_sh)

        if with_cnt:
            pltpu.sync_copy(pat_hbm, msg_v)
        plsc.subcore_barrier()

        row0 = wid * rows_per_tile
        for c in range(n_chunks):
            rbase = row0 + c * CHUNK_ROWS
            pltpu.sync_copy(src_hbm.at[pl.ds(rbase, CHUNK_ROWS)], src_v)
            pltpu.sync_copy(dst_hbm.at[pl.ds(rbase, CHUNK_ROWS)], dst_v)
            pltpu.sync_copy(p_hbm.at[pl.ds(rbase * SUB, c_edges)], p_v)
            copies = [
                pltpu.async_copy(t_hbm.at[src_v.at[j]],
                                 rows_v.at[pl.ds(j * SUB, SUB)], sem)
                for j in range(CHUNK_ROWS)
            ]
            for cp in copies:
                cp.wait()

            def grp(g, carry):
                base = g * LANES
                p_vec = p_v[pl.ds(base, LANES)]
                for e16 in range(LANES):
                    row = base + e16
                    pb = lax.broadcast(p_vec[e16], (LANES,))
                    u = rows_v[row, pl.ds(0, feat)]
                    dv = rows_v[row, pl.ds(feat, feat)]
                    msg_v[row, pl.ds(0, feat)] = u + pb * dv
                return carry

            lax.fori_loop(0, c_edges // LANES, grp, 0)

            for j in range(CHUNK_ROWS):
                pltpu.sync_copy(msg_v.at[pl.ds(j * SUB, SUB)],
                                agg_sh.at[dst_v.at[j]], add=True)

        plsc.subcore_barrier()
        r0 = sid * n_read
        pltpu.sync_copy(agg_sh.at[pl.ds(r0, n_read)],
                        agg_out.at[pl.ds(cid * n_nodes + r0, n_read)])

    return edge_kernel


def kernel(x, edge_index, edge_attr, W1, root1, b1, W2, root2, b2):
    n, f_in = x.shape
    e = edge_index.shape[1]
    hid = W1.shape[2]
    ncls = W2.shape[2]
    bn = 1000

    n_pad = ((n + 127) // 128) * 128
    rpt = ((e + SUB * NWORKERS * CHUNK_ROWS - 1) // (SUB * NWORKERS * CHUNK_ROWS)) * CHUNK_ROWS
    e_pad = rpt * NWORKERS * SUB
    padn = e_pad - e
    src2 = jnp.concatenate([edge_index[0], jnp.zeros((padn,), jnp.int32)]).reshape(e_pad // SUB, SUB)
    dst2 = jnp.concatenate([edge_index[1], jnp.full((padn,), n, jnp.int32)]).reshape(e_pad // SUB, SUB)
    p = jnp.concatenate([edge_attr[:, 0], jnp.zeros((padn,), jnp.float32)])

    wud1 = jnp.concatenate([W1[0], W1[1] - W1[0]], axis=1)
    wud2 = jnp.concatenate([W2[0], W2[1] - W2[0]], axis=1)
    zf1 = jnp.zeros((n_pad, hid + 16), jnp.float32)
    zf2 = jnp.zeros((n_pad, ncls), jnp.float32)
    c_edges = CHUNK_ROWS * SUB
    pat = jnp.zeros((c_edges, hid + 16), jnp.float32).at[:, hid].set(1.0)

    t1, r1 = _tc_proj(x, wud1, root1, b1.reshape(1, hid), bn)

    edge1 = _make_edge_kernel(n_pad, e_pad // SUB, hid, with_cnt=True)
    aggp1 = edge1(t1, src2, dst2, p, zf1, pat).reshape(2, n_pad, hid + 16)

    t2, r2 = _tc_mid(aggp1, r1, wud2, root2, b2.reshape(1, ncls), bn)

    edge2 = _make_edge_kernel(n_pad, e_pad // SUB, ncls, with_cnt=False)
    aggp2 = edge2(t2, src2, dst2, p, zf2)
    aggp2 = aggp2.reshape(2, n_pad, ncls)

    cntp = aggp1[:, :, hid:hid + 1]
    return _tc_out(aggp2, cntp, r2, bn)


# speedup vs baseline: 11.0075x; 1.2344x over previous
import functools

import jax
import jax.numpy as jnp
from jax import lax
from jax.experimental import pallas as pl
from jax.experimental.pallas import tpu as pltpu
from jax.experimental.pallas import tpu_sc as plsc

SUB = 64
CHUNK_ROWS = 8
NWORKERS = 32
LANES = 16


def _proj_body(x_ref, wud_ref, wr_ref, b_ref, t_ref, r_ref):
    x = x_ref[...]
    t_ref[...] = jnp.dot(x, wud_ref[...], preferred_element_type=jnp.float32)
    r_ref[...] = jnp.dot(x, wr_ref[...], preferred_element_type=jnp.float32) + b_ref[...]


def _mid_body(aggp_ref, r1_ref, wud_ref, wr_ref, b_ref, t_ref, r_ref):
    a = aggp_ref[...]
    s = a[0] + a[1]
    mean = s[:, :16] / jnp.maximum(s[:, 16:17], 1.0)
    t = mean + r1_ref[...]
    h = jnp.where(t > 0.0, t, jnp.exp(jnp.minimum(t, 0.0)) - 1.0)
    t_ref[...] = jnp.dot(h, wud_ref[...], preferred_element_type=jnp.float32)
    r_ref[...] = jnp.dot(h, wr_ref[...], preferred_element_type=jnp.float32) + b_ref[...]


def _out_body(aggp_ref, cntp_ref, r2_ref, o_ref):
    a = aggp_ref[...]
    c = cntp_ref[...]
    y = (a[0] + a[1]) / jnp.maximum(c[0] + c[1], 1.0) + r2_ref[...]
    m = jnp.max(y, axis=1, keepdims=True)
    e = y - m
    lse = jnp.log(jnp.sum(jnp.exp(e), axis=1, keepdims=True))
    o_ref[...] = e - lse


def _tc_proj(x, wud, wr, brow, bn):
    n, fin = x.shape
    fo = wud.shape[1]
    fr = wr.shape[1]
    grid = n // bn
    return pl.pallas_call(
        _proj_body,
        grid=(grid,),
        in_specs=[
            pl.BlockSpec((bn, fin), lambda i: (i, 0)),
            pl.BlockSpec((fin, fo), lambda i: (0, 0)),
            pl.BlockSpec((fin, fr), lambda i: (0, 0)),
            pl.BlockSpec((1, fr), lambda i: (0, 0)),
        ],
        out_specs=[
            pl.BlockSpec((bn, fo), lambda i: (i, 0)),
            pl.BlockSpec((bn, fr), lambda i: (i, 0)),
        ],
        out_shape=[
            jax.ShapeDtypeStruct((n, fo), jnp.float32),
            jax.ShapeDtypeStruct((n, fr), jnp.float32),
        ],
    )(x, wud, wr, brow)


def _tc_mid(aggp, r1, wud, wr, brow, bn):
    n, f = r1.shape
    fa = aggp.shape[2]
    fo = wud.shape[1]
    fr = wr.shape[1]
    grid = n // bn
    return pl.pallas_call(
        _mid_body,
        grid=(grid,),
        in_specs=[
            pl.BlockSpec((2, bn, fa), lambda i: (0, i, 0)),
            pl.BlockSpec((bn, f), lambda i: (i, 0)),
            pl.BlockSpec((f, fo), lambda i: (0, 0)),
            pl.BlockSpec((f, fr), lambda i: (0, 0)),
            pl.BlockSpec((1, fr), lambda i: (0, 0)),
        ],
        out_specs=[
            pl.BlockSpec((bn, fo), lambda i: (i, 0)),
            pl.BlockSpec((bn, fr), lambda i: (i, 0)),
        ],
        out_shape=[
            jax.ShapeDtypeStruct((n, fo), jnp.float32),
            jax.ShapeDtypeStruct((n, fr), jnp.float32),
        ],
    )(aggp, r1, wud, wr, brow)


def _tc_out(aggp, cntp, r2, bn):
    n, f = r2.shape
    grid = n // bn
    return pl.pallas_call(
        _out_body,
        grid=(grid,),
        in_specs=[
            pl.BlockSpec((2, bn, f), lambda i: (0, i, 0)),
            pl.BlockSpec((2, bn, 1), lambda i: (0, i, 0)),
            pl.BlockSpec((bn, f), lambda i: (i, 0)),
        ],
        out_specs=pl.BlockSpec((bn, f), lambda i: (i, 0)),
        out_shape=jax.ShapeDtypeStruct((n, f), jnp.float32),
    )(aggp, cntp, r2)


def _make_edge_kernel(n_nodes, n_idx_rows, feat, with_cnt):
    rows_per_tile = n_idx_rows // NWORKERS
    n_chunks = rows_per_tile // CHUNK_ROWS
    c_edges = CHUNK_ROWS * SUB
    n_read = n_nodes // LANES
    fw = feat + 16 if with_cnt else feat
    mesh = plsc.VectorSubcoreMesh(core_axis_name="c", subcore_axis_name="s")

    out_type = jax.ShapeDtypeStruct((2 * n_nodes, fw), jnp.float32)
    scratch = [
        pltpu.VMEM((2, CHUNK_ROWS, SUB), jnp.int32),
        pltpu.VMEM((3, CHUNK_ROWS, SUB), jnp.int32),
        pltpu.VMEM((2, c_edges), jnp.float32),
        pltpu.VMEM((2, c_edges, 2 * feat), jnp.float32),
        pltpu.VMEM((3, c_edges, fw), jnp.float32),
        pltpu.VMEM_SHARED((n_nodes, fw), jnp.float32),
        pltpu.SemaphoreType.DMA,
        pltpu.SemaphoreType.DMA,
        pltpu.SemaphoreType.DMA,
    ]

    @functools.partial(pl.kernel, mesh=mesh, out_type=out_type,
                       scratch_types=scratch,
                       compiler_params=pltpu.CompilerParams(use_tc_tiling_on_sc=False))
    def edge_kernel(*refs):
        if with_cnt:
            (t_hbm, src_hbm, dst_hbm, p_hbm, zf_hbm, pat_hbm,
             agg_out,
             src_v, dst_v, p_v, rows_v, msg_v, agg_sh,
             sem_g, sem_i, sem_s) = refs
        else:
            (t_hbm, src_hbm, dst_hbm, p_hbm, zf_hbm,
             agg_out,
             src_v, dst_v, p_v, rows_v, msg_v, agg_sh,
             sem_g, sem_i, sem_s) = refs
        cid = lax.axis_index("c")
        sid = lax.axis_index("s")
        wid = cid * 16 + sid

        @pl.when(sid == 0)
        def _():
            pltpu.sync_copy(zf_hbm, agg_sh)

        if with_cnt:
            for b in range(3):
                pltpu.sync_copy(pat_hbm, msg_v.at[b])
        plsc.subcore_barrier()

        row0 = wid * rows_per_tile

        def issue_sp(i):
            b = i % 2
            rbase = row0 + i * CHUNK_ROWS
            return [
                pltpu.async_copy(src_hbm.at[pl.ds(rbase, CHUNK_ROWS)],
                                 src_v.at[b], sem_i),
                pltpu.async_copy(p_hbm.at[pl.ds(rbase * SUB, c_edges)],
                                 p_v.at[b], sem_i),
            ]

        def issue_dst(i):
            b = i % 3
            rbase = row0 + i * CHUNK_ROWS
            return [pltpu.async_copy(dst_hbm.at[pl.ds(rbase, CHUNK_ROWS)],
                                     dst_v.at[b], sem_i)]

        def issue_gathers(i):
            b = i % 2
            return [
                pltpu.async_copy(t_hbm.at[src_v.at[b, j]],
                                 rows_v.at[b, pl.ds(j * SUB, SUB)], sem_g)
                for j in range(CHUNK_ROWS)
            ]

        def issue_scatters(i):
            b = i % 3
            return [
                pltpu.async_copy(msg_v.at[b, pl.ds(j * SUB, SUB)],
                                 agg_sh.at[dst_v.at[b, j]], sem_s, add=True)
                for j in range(CHUNK_ROWS)
            ]

        def compute(i):
            b2 = i % 2
            b3 = i % 3

            def grp(g, carry):
                base = g * LANES
                p_vec = p_v[b2, pl.ds(base, LANES)]
                for e16 in range(LANES):
                    row = base + e16
                    pb = lax.broadcast(p_vec[e16], (LANES,))
                    u = rows_v[b2, row, pl.ds(0, feat)]
                    dv = rows_v[b2, row, pl.ds(feat, feat)]
                    msg_v[b3, row, pl.ds(0, feat)] = u + pb * dv
                return carry

            lax.fori_loop(0, c_edges // LANES, grp, 0)

        def drain(cps):
            for cp in cps:
                cp.wait()

        pend_sp = {0: issue_sp(0)}
        if n_chunks > 1:
            pend_sp[1] = issue_sp(1)
        pend_dst = {0: issue_dst(0)}
        pend_g = {}
        pend_s = {}
        drain(pend_sp.pop(0))
        pend_g[0] = issue_gathers(0)
        for i in range(n_chunks):
            if i + 1 < n_chunks:
                drain(pend_sp.pop(i + 1))
            drain(pend_g.pop(i))
            if i + 1 < n_chunks:
                pend_g[i + 1] = issue_gathers(i + 1)
            if i >= 3 and (i - 3) in pend_s:
                drain(pend_s.pop(i - 3))
            if i + 1 < n_chunks:
                if i >= 2 and (i - 2) in pend_s:
                    drain(pend_s.pop(i - 2))
                pend_dst[i + 1] = issue_dst(i + 1)
            drain(pend_dst.pop(i))
            compute(i)
            pend_s[i] = issue_scatters(i)
            if i + 2 < n_chunks:
                pend_sp[i + 2] = issue_sp(i + 2)
        for i in sorted(pend_s):
            drain(pend_s[i])

        plsc.subcore_barrier()
        r0 = sid * n_read
        pltpu.sync_copy(agg_sh.at[pl.ds(r0, n_read)],
                        agg_out.at[pl.ds(cid * n_nodes + r0, n_read)])

    return edge_kernel


def kernel(x, edge_index, edge_attr, W1, root1, b1, W2, root2, b2):
    n, f_in = x.shape
    e = edge_index.shape[1]
    hid = W1.shape[2]
    ncls = W2.shape[2]
    bn = 1000

    n_pad = ((n + 127) // 128) * 128
    rpt = ((e + SUB * NWORKERS * CHUNK_ROWS - 1) // (SUB * NWORKERS * CHUNK_ROWS)) * CHUNK_ROWS
    e_pad = rpt * NWORKERS * SUB
    padn = e_pad - e
    src2 = jnp.concatenate([edge_index[0], jnp.zeros((padn,), jnp.int32)]).reshape(e_pad // SUB, SUB)
    dst2 = jnp.concatenate([edge_index[1], jnp.full((padn,), n, jnp.int32)]).reshape(e_pad // SUB, SUB)
    p = jnp.concatenate([edge_attr[:, 0], jnp.zeros((padn,), jnp.float32)])

    wud1 = jnp.concatenate([W1[0], W1[1] - W1[0]], axis=1)
    wud2 = jnp.concatenate([W2[0], W2[1] - W2[0]], axis=1)
    zf1 = jnp.zeros((n_pad, hid + 16), jnp.float32)
    zf2 = jnp.zeros((n_pad, ncls), jnp.float32)
    c_edges = CHUNK_ROWS * SUB
    pat = jnp.zeros((c_edges, hid + 16), jnp.float32).at[:, hid].set(1.0)

    t1, r1 = _tc_proj(x, wud1, root1, b1.reshape(1, hid), bn)

    edge1 = _make_edge_kernel(n_pad, e_pad // SUB, hid, with_cnt=True)
    aggp1 = edge1(t1, src2, dst2, p, zf1, pat).reshape(2, n_pad, hid + 16)

    t2, r2 = _tc_mid(aggp1, r1, wud2, root2, b2.reshape(1, ncls), bn)

    edge2 = _make_edge_kernel(n_pad, e_pad // SUB, ncls, with_cnt=False)
    aggp2 = edge2(t2, src2, dst2, p, zf2)
    aggp2 = aggp2.reshape(2, n_pad, ncls)

    cntp = aggp1[:, :, hid:hid + 1]
    return _tc_out(aggp2, cntp, r2, bn)


# speedup vs baseline: 11.4369x; 1.0390x over previous
import functools

import jax
import jax.numpy as jnp
from jax import lax
from jax.experimental import pallas as pl
from jax.experimental.pallas import tpu as pltpu
from jax.experimental.pallas import tpu_sc as plsc

SUB = 128
CHUNK_ROWS = 8
NWORKERS = 32
LANES = 16


def _proj_body(x_ref, wud_ref, wr_ref, b_ref, t_ref, r_ref):
    x = x_ref[...]
    t_ref[...] = jnp.dot(x, wud_ref[...], preferred_element_type=jnp.float32)
    r_ref[...] = jnp.dot(x, wr_ref[...], preferred_element_type=jnp.float32) + b_ref[...]


def _mid_body(aggp_ref, r1_ref, wud_ref, wr_ref, b_ref, t_ref, r_ref):
    a = aggp_ref[...]
    s = a[0] + a[1]
    mean = s[:, :16] / jnp.maximum(s[:, 16:17], 1.0)
    t = mean + r1_ref[...]
    h = jnp.where(t > 0.0, t, jnp.exp(jnp.minimum(t, 0.0)) - 1.0)
    t_ref[...] = jnp.dot(h, wud_ref[...], preferred_element_type=jnp.float32)
    r_ref[...] = jnp.dot(h, wr_ref[...], preferred_element_type=jnp.float32) + b_ref[...]


def _out_body(aggp_ref, cntp_ref, r2_ref, o_ref):
    a = aggp_ref[...]
    c = cntp_ref[...]
    y = (a[0] + a[1]) / jnp.maximum(c[0] + c[1], 1.0) + r2_ref[...]
    m = jnp.max(y, axis=1, keepdims=True)
    e = y - m
    lse = jnp.log(jnp.sum(jnp.exp(e), axis=1, keepdims=True))
    o_ref[...] = e - lse


def _tc_proj(x, wud, wr, brow, bn):
    n, fin = x.shape
    fo = wud.shape[1]
    fr = wr.shape[1]
    grid = n // bn
    return pl.pallas_call(
        _proj_body,
        grid=(grid,),
        in_specs=[
            pl.BlockSpec((bn, fin), lambda i: (i, 0)),
            pl.BlockSpec((fin, fo), lambda i: (0, 0)),
            pl.BlockSpec((fin, fr), lambda i: (0, 0)),
            pl.BlockSpec((1, fr), lambda i: (0, 0)),
        ],
        out_specs=[
            pl.BlockSpec((bn, fo), lambda i: (i, 0)),
            pl.BlockSpec((bn, fr), lambda i: (i, 0)),
        ],
        out_shape=[
            jax.ShapeDtypeStruct((n, fo), jnp.float32),
            jax.ShapeDtypeStruct((n, fr), jnp.float32),
        ],
    )(x, wud, wr, brow)


def _tc_mid(aggp, r1, wud, wr, brow, bn):
    n, f = r1.shape
    fa = aggp.shape[2]
    fo = wud.shape[1]
    fr = wr.shape[1]
    grid = n // bn
    return pl.pallas_call(
        _mid_body,
        grid=(grid,),
        in_specs=[
            pl.BlockSpec((2, bn, fa), lambda i: (0, i, 0)),
            pl.BlockSpec((bn, f), lambda i: (i, 0)),
            pl.BlockSpec((f, fo), lambda i: (0, 0)),
            pl.BlockSpec((f, fr), lambda i: (0, 0)),
            pl.BlockSpec((1, fr), lambda i: (0, 0)),
        ],
        out_specs=[
            pl.BlockSpec((bn, fo), lambda i: (i, 0)),
            pl.BlockSpec((bn, fr), lambda i: (i, 0)),
        ],
        out_shape=[
            jax.ShapeDtypeStruct((n, fo), jnp.float32),
            jax.ShapeDtypeStruct((n, fr), jnp.float32),
        ],
    )(aggp, r1, wud, wr, brow)


def _tc_out(aggp, cntp, r2, bn):
    n, f = r2.shape
    grid = n // bn
    return pl.pallas_call(
        _out_body,
        grid=(grid,),
        in_specs=[
            pl.BlockSpec((2, bn, f), lambda i: (0, i, 0)),
            pl.BlockSpec((2, bn, 1), lambda i: (0, i, 0)),
            pl.BlockSpec((bn, f), lambda i: (i, 0)),
        ],
        out_specs=pl.BlockSpec((bn, f), lambda i: (i, 0)),
        out_shape=jax.ShapeDtypeStruct((n, f), jnp.float32),
    )(aggp, cntp, r2)


def _make_edge_kernel(n_nodes, n_idx_rows, feat, with_cnt):
    rows_per_tile = n_idx_rows // NWORKERS
    n_chunks = rows_per_tile // CHUNK_ROWS
    c_edges = CHUNK_ROWS * SUB
    n_read = n_nodes // LANES
    fw = feat + 16 if with_cnt else feat
    mesh = plsc.VectorSubcoreMesh(core_axis_name="c", subcore_axis_name="s")

    out_type = jax.ShapeDtypeStruct((2 * n_nodes, fw), jnp.float32)
    scratch = [
        pltpu.VMEM((2, CHUNK_ROWS, SUB), jnp.int32),
        pltpu.VMEM((3, CHUNK_ROWS, SUB), jnp.int32),
        pltpu.VMEM((2, c_edges), jnp.float32),
        pltpu.VMEM((2, c_edges, 2 * feat), jnp.float32),
        pltpu.VMEM((c_edges, fw), jnp.float32),
        pltpu.VMEM_SHARED((n_nodes, fw), jnp.float32),
        pltpu.SemaphoreType.DMA,
        pltpu.SemaphoreType.DMA,
        pltpu.SemaphoreType.DMA,
    ]

    @functools.partial(pl.kernel, mesh=mesh, out_type=out_type,
                       scratch_types=scratch,
                       compiler_params=pltpu.CompilerParams(use_tc_tiling_on_sc=False))
    def edge_kernel(*refs):
        if with_cnt:
            (t_hbm, src_hbm, dst_hbm, p_hbm, zf_hbm, pat_hbm,
             agg_out,
             src_v, dst_v, p_v, rows_v, msg_v, agg_sh,
             sem_g, sem_i, sem_s) = refs
        else:
            (t_hbm, src_hbm, dst_hbm, p_hbm, zf_hbm,
             agg_out,
             src_v, dst_v, p_v, rows_v, msg_v, agg_sh,
             sem_g, sem_i, sem_s) = refs
        cid = lax.axis_index("c")
        sid = lax.axis_index("s")
        wid = cid * 16 + sid

        @pl.when(sid == 0)
        def _():
            pltpu.sync_copy(zf_hbm, agg_sh)

        if with_cnt:
            pltpu.sync_copy(pat_hbm, msg_v)
        plsc.subcore_barrier()

        row0 = wid * rows_per_tile

        def issue_sp(i):
            b = i % 2
            rbase = row0 + i * CHUNK_ROWS
            return [
                pltpu.async_copy(src_hbm.at[pl.ds(rbase, CHUNK_ROWS)],
                                 src_v.at[b], sem_i),
                pltpu.async_copy(p_hbm.at[pl.ds(rbase * SUB, c_edges)],
                                 p_v.at[b], sem_i),
            ]

        def issue_dst(i):
            b = i % 3
            rbase = row0 + i * CHUNK_ROWS
            return [pltpu.async_copy(dst_hbm.at[pl.ds(rbase, CHUNK_ROWS)],
                                     dst_v.at[b], sem_i)]

        def issue_gathers(i):
            b = i % 2
            return [
                pltpu.async_copy(t_hbm.at[src_v.at[b, j]],
                                 rows_v.at[b, pl.ds(j * SUB, SUB)], sem_g)
                for j in range(CHUNK_ROWS)
            ]

        def issue_scatters(i):
            b = i % 3
            return [
                pltpu.async_copy(msg_v.at[pl.ds(j * SUB, SUB)],
                                 agg_sh.at[dst_v.at[b, j]], sem_s, add=True)
                for j in range(CHUNK_ROWS)
            ]

        def compute(i):
            b2 = i % 2

            def grp(g, carry):
                base = g * LANES
                p_vec = p_v[b2, pl.ds(base, LANES)]
                for e16 in range(LANES):
                    row = base + e16
                    pb = lax.broadcast(p_vec[e16], (LANES,))
                    u = rows_v[b2, row, pl.ds(0, feat)]
                    dv = rows_v[b2, row, pl.ds(feat, feat)]
                    msg_v[row, pl.ds(0, feat)] = u + pb * dv
                return carry

            lax.fori_loop(0, c_edges // LANES, grp, 0)

        def drain(cps):
            for cp in cps:
                cp.wait()

        pend_sp = {0: issue_sp(0)}
        if n_chunks > 1:
            pend_sp[1] = issue_sp(1)
        pend_dst = {0: issue_dst(0)}
        pend_g = {}
        pend_s = {}
        drain(pend_sp.pop(0))
        pend_g[0] = issue_gathers(0)
        for i in range(n_chunks):
            if i + 1 < n_chunks:
                drain(pend_sp.pop(i + 1))
            drain(pend_g.pop(i))
            if i + 1 < n_chunks:
                pend_g[i + 1] = issue_gathers(i + 1)
            if i >= 1 and (i - 1) in pend_s:
                drain(pend_s.pop(i - 1))
            if i + 1 < n_chunks:
                pend_dst[i + 1] = issue_dst(i + 1)
            drain(pend_dst.pop(i))
            compute(i)
            pend_s[i] = issue_scatters(i)
            if i + 2 < n_chunks:
                pend_sp[i + 2] = issue_sp(i + 2)
        for i in sorted(pend_s):
            drain(pend_s[i])

        plsc.subcore_barrier()
        r0 = sid * n_read
        pltpu.sync_copy(agg_sh.at[pl.ds(r0, n_read)],
                        agg_out.at[pl.ds(cid * n_nodes + r0, n_read)])

    return edge_kernel


def kernel(x, edge_index, edge_attr, W1, root1, b1, W2, root2, b2):
    n, f_in = x.shape
    e = edge_index.shape[1]
    hid = W1.shape[2]
    ncls = W2.shape[2]
    bn = 1000

    n_pad = ((n + 127) // 128) * 128
    rpt = ((e + SUB * NWORKERS * CHUNK_ROWS - 1) // (SUB * NWORKERS * CHUNK_ROWS)) * CHUNK_ROWS
    e_pad = rpt * NWORKERS * SUB
    padn = e_pad - e
    src2 = jnp.concatenate([edge_index[0], jnp.zeros((padn,), jnp.int32)]).reshape(e_pad // SUB, SUB)
    dst2 = jnp.concatenate([edge_index[1], jnp.full((padn,), n, jnp.int32)]).reshape(e_pad // SUB, SUB)
    p = jnp.concatenate([edge_attr[:, 0], jnp.zeros((padn,), jnp.float32)])

    wud1 = jnp.concatenate([W1[0], W1[1] - W1[0]], axis=1)
    wud2 = jnp.concatenate([W2[0], W2[1] - W2[0]], axis=1)
    zf1 = jnp.zeros((n_pad, hid + 16), jnp.float32)
    zf2 = jnp.zeros((n_pad, ncls), jnp.float32)
    c_edges = CHUNK_ROWS * SUB
    pat = jnp.zeros((c_edges, hid + 16), jnp.float32).at[:, hid].set(1.0)

    t1, r1 = _tc_proj(x, wud1, root1, b1.reshape(1, hid), bn)

    edge1 = _make_edge_kernel(n_pad, e_pad // SUB, hid, with_cnt=True)
    aggp1 = edge1(t1, src2, dst2, p, zf1, pat).reshape(2, n_pad, hid + 16)

    t2, r2 = _tc_mid(aggp1, r1, wud2, root2, b2.reshape(1, ncls), bn)

    edge2 = _make_edge_kernel(n_pad, e_pad // SUB, ncls, with_cnt=False)
    aggp2 = edge2(t2, src2, dst2, p, zf2)
    aggp2 = aggp2.reshape(2, n_pad, ncls)

    cntp = aggp1[:, :, hid:hid + 1]
    return _tc_out(aggp2, cntp, r2, bn)


# speedup vs baseline: 15.4159x; 1.3479x over previous
import functools

import jax
import jax.numpy as jnp
from jax import lax
from jax.experimental import pallas as pl
from jax.experimental.pallas import tpu as pltpu
from jax.experimental.pallas import tpu_sc as plsc

SUB = 64
CHUNK_ROWS = 8
NWORKERS = 32
LANES = 16


def _proj_body(x_ref, wud_ref, wr_ref, b_ref, t_ref, r_ref):
    x = x_ref[...]
    t_ref[...] = jnp.dot(x, wud_ref[...], preferred_element_type=jnp.float32)
    r_ref[...] = jnp.dot(x, wr_ref[...], preferred_element_type=jnp.float32) + b_ref[...]


def _mid_body(aggp_ref, r1_ref, wud_ref, wr_ref, b_ref, t_ref, r_ref):
    a = aggp_ref[...]
    s = a[0] + a[1]
    mean = s[:, :16] / jnp.maximum(s[:, 16:17], 1.0)
    t = mean + r1_ref[...]
    h = jnp.where(t > 0.0, t, jnp.exp(jnp.minimum(t, 0.0)) - 1.0)
    t_ref[...] = jnp.dot(h, wud_ref[...], preferred_element_type=jnp.float32)
    r_ref[...] = jnp.dot(h, wr_ref[...], preferred_element_type=jnp.float32) + b_ref[...]


def _out_body(aggp_ref, cntp_ref, r2_ref, o_ref):
    a = aggp_ref[...]
    c = cntp_ref[...]
    y = (a[0] + a[1]) / jnp.maximum(c[0] + c[1], 1.0) + r2_ref[...]
    m = jnp.max(y, axis=1, keepdims=True)
    e = y - m
    lse = jnp.log(jnp.sum(jnp.exp(e), axis=1, keepdims=True))
    o_ref[...] = e - lse


def _tc_proj(x, wud, wr, brow, bn):
    n, fin = x.shape
    fo = wud.shape[1]
    fr = wr.shape[1]
    grid = n // bn
    return pl.pallas_call(
        _proj_body,
        grid=(grid,),
        in_specs=[
            pl.BlockSpec((bn, fin), lambda i: (i, 0)),
            pl.BlockSpec((fin, fo), lambda i: (0, 0)),
            pl.BlockSpec((fin, fr), lambda i: (0, 0)),
            pl.BlockSpec((1, fr), lambda i: (0, 0)),
        ],
        out_specs=[
            pl.BlockSpec((bn, fo), lambda i: (i, 0)),
            pl.BlockSpec((bn, fr), lambda i: (i, 0)),
        ],
        out_shape=[
            jax.ShapeDtypeStruct((n, fo), jnp.float32),
            jax.ShapeDtypeStruct((n, fr), jnp.float32),
        ],
    )(x, wud, wr, brow)


def _tc_mid(aggp, r1, wud, wr, brow, bn):
    n, f = r1.shape
    fa = aggp.shape[2]
    fo = wud.shape[1]
    fr = wr.shape[1]
    grid = n // bn
    return pl.pallas_call(
        _mid_body,
        grid=(grid,),
        in_specs=[
            pl.BlockSpec((2, bn, fa), lambda i: (0, i, 0)),
            pl.BlockSpec((bn, f), lambda i: (i, 0)),
            pl.BlockSpec((f, fo), lambda i: (0, 0)),
            pl.BlockSpec((f, fr), lambda i: (0, 0)),
            pl.BlockSpec((1, fr), lambda i: (0, 0)),
        ],
        out_specs=[
            pl.BlockSpec((bn, fo), lambda i: (i, 0)),
            pl.BlockSpec((bn, fr), lambda i: (i, 0)),
        ],
        out_shape=[
            jax.ShapeDtypeStruct((n, fo), jnp.float32),
            jax.ShapeDtypeStruct((n, fr), jnp.float32),
        ],
    )(aggp, r1, wud, wr, brow)


def _tc_out(aggp, cntp, r2, bn):
    n, f = r2.shape
    grid = n // bn
    return pl.pallas_call(
        _out_body,
        grid=(grid,),
        in_specs=[
            pl.BlockSpec((2, bn, f), lambda i: (0, i, 0)),
            pl.BlockSpec((2, bn, 1), lambda i: (0, i, 0)),
            pl.BlockSpec((bn, f), lambda i: (i, 0)),
        ],
        out_specs=pl.BlockSpec((bn, f), lambda i: (i, 0)),
        out_shape=jax.ShapeDtypeStruct((n, f), jnp.float32),
    )(aggp, cntp, r2)


def _make_edge_kernel(n_nodes, n_idx_rows, feat, with_cnt):
    rows_per_tile = n_idx_rows // NWORKERS
    n_chunks = rows_per_tile // CHUNK_ROWS
    c_edges = CHUNK_ROWS * SUB
    n_read = n_nodes // LANES
    fw = feat + 16 if with_cnt else feat
    mesh = plsc.VectorSubcoreMesh(core_axis_name="c", subcore_axis_name="s")

    out_type = jax.ShapeDtypeStruct((2 * n_nodes, fw), jnp.float32)
    scratch = [
        pltpu.VMEM((2, CHUNK_ROWS, SUB), jnp.int32),
        pltpu.VMEM((3, CHUNK_ROWS, SUB), jnp.int32),
        pltpu.VMEM((2, c_edges), jnp.float32),
        pltpu.VMEM((2, c_edges, 2 * feat), jnp.float32),
        pltpu.VMEM((c_edges, fw), jnp.float32),
        pltpu.VMEM_SHARED((n_nodes, fw), jnp.float32),
        pltpu.VMEM_SHARED((n_nodes, 2 * feat), jnp.float32),
        pltpu.SemaphoreType.DMA,
        pltpu.SemaphoreType.DMA,
        pltpu.SemaphoreType.DMA,
    ]

    @functools.partial(pl.kernel, mesh=mesh, out_type=out_type,
                       scratch_types=scratch,
                       compiler_params=pltpu.CompilerParams(use_tc_tiling_on_sc=False))
    def edge_kernel(*refs):
        if with_cnt:
            (t_hbm, src_hbm, dst_hbm, p_hbm, zf_hbm, pat_hbm,
             agg_out,
             src_v, dst_v, p_v, rows_v, msg_v, agg_sh, t_sh,
             sem_g, sem_i, sem_s) = refs
        else:
            (t_hbm, src_hbm, dst_hbm, p_hbm, zf_hbm,
             agg_out,
             src_v, dst_v, p_v, rows_v, msg_v, agg_sh, t_sh,
             sem_g, sem_i, sem_s) = refs
        cid = lax.axis_index("c")
        sid = lax.axis_index("s")
        wid = cid * 16 + sid

        @pl.when(sid == 0)
        def _():
            pltpu.sync_copy(zf_hbm, agg_sh)

        @pl.when(sid == 1)
        def _():
            n_tr = t_hbm.shape[0]
            pltpu.sync_copy(t_hbm, t_sh.at[pl.ds(0, n_tr)])

        if with_cnt:
            pltpu.sync_copy(pat_hbm, msg_v)
        plsc.subcore_barrier()

        row0 = wid * rows_per_tile

        def issue_sp(i):
            b = i % 2
            rbase = row0 + i * CHUNK_ROWS
            return [
                pltpu.async_copy(src_hbm.at[pl.ds(rbase, CHUNK_ROWS)],
                                 src_v.at[b], sem_i),
                pltpu.async_copy(p_hbm.at[pl.ds(rbase * SUB, c_edges)],
                                 p_v.at[b], sem_i),
            ]

        def issue_dst(i):
            b = i % 3
            rbase = row0 + i * CHUNK_ROWS
            return [pltpu.async_copy(dst_hbm.at[pl.ds(rbase, CHUNK_ROWS)],
                                     dst_v.at[b], sem_i)]

        def issue_gathers(i):
            b = i % 2
            return [
                pltpu.async_copy(t_sh.at[src_v.at[b, j]],
                                 rows_v.at[b, pl.ds(j * SUB, SUB)], sem_g)
                for j in range(CHUNK_ROWS)
            ]

        def issue_scatters(i):
            b = i % 3
            return [
                pltpu.async_copy(msg_v.at[pl.ds(j * SUB, SUB)],
                                 agg_sh.at[dst_v.at[b, j]], sem_s, add=True)
                for j in range(CHUNK_ROWS)
            ]

        def compute(i):
            b2 = i % 2

            def grp(g, carry):
                base = g * LANES
                p_vec = p_v[b2, pl.ds(base, LANES)]
                for e16 in range(LANES):
                    row = base + e16
                    pb = lax.broadcast(p_vec[e16], (LANES,))
                    u = rows_v[b2, row, pl.ds(0, feat)]
                    dv = rows_v[b2, row, pl.ds(feat, feat)]
                    msg_v[row, pl.ds(0, feat)] = u + pb * dv
                return carry

            lax.fori_loop(0, c_edges // LANES, grp, 0)

        def drain(cps):
            for cp in cps:
                cp.wait()

        pend_sp = {0: issue_sp(0)}
        if n_chunks > 1:
            pend_sp[1] = issue_sp(1)
        pend_dst = {0: issue_dst(0)}
        pend_g = {}
        pend_s = {}
        drain(pend_sp.pop(0))
        pend_g[0] = issue_gathers(0)
        for i in range(n_chunks):
            if i + 1 < n_chunks:
                drain(pend_sp.pop(i + 1))
            drain(pend_g.pop(i))
            if i + 1 < n_chunks:
                pend_g[i + 1] = issue_gathers(i + 1)
            if i >= 1 and (i - 1) in pend_s:
                drain(pend_s.pop(i - 1))
            if i + 1 < n_chunks:
                pend_dst[i + 1] = issue_dst(i + 1)
            drain(pend_dst.pop(i))
            compute(i)
            pend_s[i] = issue_scatters(i)
            if i + 2 < n_chunks:
                pend_sp[i + 2] = issue_sp(i + 2)
        for i in sorted(pend_s):
            drain(pend_s[i])

        plsc.subcore_barrier()
        r0 = sid * n_read
        pltpu.sync_copy(agg_sh.at[pl.ds(r0, n_read)],
                        agg_out.at[pl.ds(cid * n_nodes + r0, n_read)])

    return edge_kernel


def kernel(x, edge_index, edge_attr, W1, root1, b1, W2, root2, b2):
    n, f_in = x.shape
    e = edge_index.shape[1]
    hid = W1.shape[2]
    ncls = W2.shape[2]
    bn = 1000

    n_pad = ((n + 127) // 128) * 128
    rpt = ((e + SUB * NWORKERS * CHUNK_ROWS - 1) // (SUB * NWORKERS * CHUNK_ROWS)) * CHUNK_ROWS
    e_pad = rpt * NWORKERS * SUB
    padn = e_pad - e
    src2 = jnp.concatenate([edge_index[0], jnp.zeros((padn,), jnp.int32)]).reshape(e_pad // SUB, SUB)
    dst2 = jnp.concatenate([edge_index[1], jnp.full((padn,), n, jnp.int32)]).reshape(e_pad // SUB, SUB)
    p = jnp.concatenate([edge_attr[:, 0], jnp.zeros((padn,), jnp.float32)])

    wud1 = jnp.concatenate([W1[0], W1[1] - W1[0]], axis=1)
    wud2 = jnp.concatenate([W2[0], W2[1] - W2[0]], axis=1)
    zf1 = jnp.zeros((n_pad, hid + 16), jnp.float32)
    zf2 = jnp.zeros((n_pad, ncls), jnp.float32)
    c_edges = CHUNK_ROWS * SUB
    pat = jnp.zeros((c_edges, hid + 16), jnp.float32).at[:, hid].set(1.0)

    t1, r1 = _tc_proj(x, wud1, root1, b1.reshape(1, hid), bn)

    edge1 = _make_edge_kernel(n_pad, e_pad // SUB, hid, with_cnt=True)
    aggp1 = edge1(t1, src2, dst2, p, zf1, pat).reshape(2, n_pad, hid + 16)

    t2, r2 = _tc_mid(aggp1, r1, wud2, root2, b2.reshape(1, ncls), bn)

    edge2 = _make_edge_kernel(n_pad, e_pad // SUB, ncls, with_cnt=False)
    aggp2 = edge2(t2, src2, dst2, p, zf2)
    aggp2 = aggp2.reshape(2, n_pad, ncls)

    cntp = aggp1[:, :, hid:hid + 1]
    return _tc_out(aggp2, cntp, r2, bn)


# speedup vs baseline: 15.5820x; 1.0108x over previous
import functools

import jax
import jax.numpy as jnp
from jax import lax
from jax.experimental import pallas as pl
from jax.experimental.pallas import tpu as pltpu
from jax.experimental.pallas import tpu_sc as plsc

SUB = 64
CHUNK_ROWS = 8
NWORKERS = 32
LANES = 16


def _proj_body(x_ref, wud_ref, wr_ref, b_ref, t_ref, r_ref):
    x = x_ref[...]
    t_ref[...] = jnp.dot(x, wud_ref[...], preferred_element_type=jnp.float32)
    r_ref[...] = jnp.dot(x, wr_ref[...], preferred_element_type=jnp.float32) + b_ref[...]


def _mid_body(aggp_ref, r1_ref, wud_ref, wr_ref, b_ref, t_ref, r_ref):
    a = aggp_ref[...]
    s = a[0] + a[1]
    mean = s[:, :16] / jnp.maximum(s[:, 16:17], 1.0)
    t = mean + r1_ref[...]
    h = jnp.where(t > 0.0, t, jnp.exp(jnp.minimum(t, 0.0)) - 1.0)
    t_ref[...] = jnp.dot(h, wud_ref[...], preferred_element_type=jnp.float32)
    r_ref[...] = jnp.dot(h, wr_ref[...], preferred_element_type=jnp.float32) + b_ref[...]


def _out_body(aggp_ref, cntp_ref, r2_ref, o_ref):
    a = aggp_ref[...]
    c = cntp_ref[...]
    y = (a[0] + a[1]) / jnp.maximum(c[0] + c[1], 1.0) + r2_ref[...]
    m = jnp.max(y, axis=1, keepdims=True)
    e = y - m
    lse = jnp.log(jnp.sum(jnp.exp(e), axis=1, keepdims=True))
    o_ref[...] = e - lse


def _tc_proj(x, wud, wr, brow, bn):
    n, fin = x.shape
    fo = wud.shape[1]
    fr = wr.shape[1]
    grid = n // bn
    return pl.pallas_call(
        _proj_body,
        grid=(grid,),
        in_specs=[
            pl.BlockSpec((bn, fin), lambda i: (i, 0)),
            pl.BlockSpec((fin, fo), lambda i: (0, 0)),
            pl.BlockSpec((fin, fr), lambda i: (0, 0)),
            pl.BlockSpec((1, fr), lambda i: (0, 0)),
        ],
        out_specs=[
            pl.BlockSpec((bn, fo), lambda i: (i, 0)),
            pl.BlockSpec((bn, fr), lambda i: (i, 0)),
        ],
        out_shape=[
            jax.ShapeDtypeStruct((n, fo), jnp.float32),
            jax.ShapeDtypeStruct((n, fr), jnp.float32),
        ],
    )(x, wud, wr, brow)


def _tc_mid(aggp, r1, wud, wr, brow, bn):
    n, f = r1.shape
    fa = aggp.shape[2]
    fo = wud.shape[1]
    fr = wr.shape[1]
    grid = n // bn
    return pl.pallas_call(
        _mid_body,
        grid=(grid,),
        in_specs=[
            pl.BlockSpec((2, bn, fa), lambda i: (0, i, 0)),
            pl.BlockSpec((bn, f), lambda i: (i, 0)),
            pl.BlockSpec((f, fo), lambda i: (0, 0)),
            pl.BlockSpec((f, fr), lambda i: (0, 0)),
            pl.BlockSpec((1, fr), lambda i: (0, 0)),
        ],
        out_specs=[
            pl.BlockSpec((bn, fo), lambda i: (i, 0)),
            pl.BlockSpec((bn, fr), lambda i: (i, 0)),
        ],
        out_shape=[
            jax.ShapeDtypeStruct((n, fo), jnp.float32),
            jax.ShapeDtypeStruct((n, fr), jnp.float32),
        ],
    )(aggp, r1, wud, wr, brow)


def _tc_out(aggp, cntp, r2, bn):
    n, f = r2.shape
    grid = n // bn
    return pl.pallas_call(
        _out_body,
        grid=(grid,),
        in_specs=[
            pl.BlockSpec((2, bn, f), lambda i: (0, i, 0)),
            pl.BlockSpec((2, bn, 1), lambda i: (0, i, 0)),
            pl.BlockSpec((bn, f), lambda i: (i, 0)),
        ],
        out_specs=pl.BlockSpec((bn, f), lambda i: (i, 0)),
        out_shape=jax.ShapeDtypeStruct((n, f), jnp.float32),
    )(aggp, cntp, r2)


def _make_edge_kernel(n_nodes, n_idx_rows, feat, with_cnt):
    rows_per_tile = n_idx_rows // NWORKERS
    n_chunks = rows_per_tile // CHUNK_ROWS
    c_edges = CHUNK_ROWS * SUB
    n_read = n_nodes // LANES
    fw = feat + 16 if with_cnt else feat
    mesh = plsc.VectorSubcoreMesh(core_axis_name="c", subcore_axis_name="s")

    out_type = jax.ShapeDtypeStruct((2 * n_nodes, fw), jnp.float32)
    scratch = [
        pltpu.VMEM((2, CHUNK_ROWS, SUB), jnp.int32),
        pltpu.VMEM((3, CHUNK_ROWS, SUB), jnp.int32),
        pltpu.VMEM((2, c_edges), jnp.float32),
        pltpu.VMEM((2, c_edges, 2 * feat), jnp.float32),
        pltpu.VMEM((c_edges, fw), jnp.float32),
        pltpu.VMEM_SHARED((n_nodes, fw), jnp.float32),
        pltpu.VMEM_SHARED((n_nodes, 2 * feat), jnp.float32),
        pltpu.SemaphoreType.DMA,
        pltpu.SemaphoreType.DMA,
        pltpu.SemaphoreType.DMA,
    ]

    @functools.partial(pl.kernel, mesh=mesh, out_type=out_type,
                       scratch_types=scratch,
                       compiler_params=pltpu.CompilerParams(use_tc_tiling_on_sc=False))
    def edge_kernel(*refs):
        if with_cnt:
            (t_hbm, src_hbm, dst_hbm, p_hbm, zf_hbm, pat_hbm,
             agg_out,
             src_v, dst_v, p_v, rows_v, msg_v, agg_sh, t_sh,
             sem_g, sem_i, sem_s) = refs
        else:
            (t_hbm, src_hbm, dst_hbm, p_hbm, zf_hbm,
             agg_out,
             src_v, dst_v, p_v, rows_v, msg_v, agg_sh, t_sh,
             sem_g, sem_i, sem_s) = refs
        cid = lax.axis_index("c")
        sid = lax.axis_index("s")
        wid = cid * 16 + sid

        @pl.when(sid == 0)
        def _():
            pltpu.sync_copy(zf_hbm, agg_sh)

        @pl.when(sid == 1)
        def _():
            n_tr = t_hbm.shape[0]
            pltpu.sync_copy(t_hbm, t_sh.at[pl.ds(0, n_tr)])

        if with_cnt:
            pltpu.sync_copy(pat_hbm, msg_v)
        plsc.subcore_barrier()

        row0 = wid * rows_per_tile

        def issue_sp(i):
            b = i % 2
            rbase = row0 + i * CHUNK_ROWS
            return [
                pltpu.async_copy(src_hbm.at[pl.ds(rbase, CHUNK_ROWS)],
                                 src_v.at[b], sem_i),
                pltpu.async_copy(p_hbm.at[pl.ds(rbase * SUB, c_edges)],
                                 p_v.at[b], sem_i),
            ]

        def issue_dst(i):
            b = i % 3
            rbase = row0 + i * CHUNK_ROWS
            return [pltpu.async_copy(dst_hbm.at[pl.ds(rbase, CHUNK_ROWS)],
                                     dst_v.at[b], sem_i)]

        def issue_gathers(i):
            b = i % 2
            return [
                pltpu.async_copy(t_sh.at[src_v.at[b, j]],
                                 rows_v.at[b, pl.ds(j * SUB, SUB)], sem_g)
                for j in range(CHUNK_ROWS)
            ]

        def issue_scatters(i):
            b = i % 3
            return [
                pltpu.async_copy(msg_v.at[pl.ds(j * SUB, SUB)],
                                 agg_sh.at[dst_v.at[b, j]], sem_s, add=True)
                for j in range(CHUNK_ROWS)
            ]

        def compute(i):
            b2 = i % 2

            def grp(g, carry):
                base = g * LANES
                p_vec = p_v[b2, pl.ds(base, LANES)]
                for e16 in range(LANES):
                    row = base + e16
                    pb = lax.broadcast(p_vec[e16], (LANES,))
                    u = rows_v[b2, row, pl.ds(0, feat)]
                    dv = rows_v[b2, row, pl.ds(feat, feat)]
                    msg_v[row, pl.ds(0, feat)] = u + pb * dv
                return carry

            lax.fori_loop(0, c_edges // LANES, grp, 0)

        def drain(cps):
            for cp in cps:
                cp.wait()

        pend_sp = {0: issue_sp(0)}
        if n_chunks > 1:
            pend_sp[1] = issue_sp(1)
        pend_dst = {0: issue_dst(0)}
        pend_g = {}
        pend_s = {}
        drain(pend_sp.pop(0))
        pend_g[0] = issue_gathers(0)
        for i in range(n_chunks):
            if i + 1 < n_chunks:
                drain(pend_sp.pop(i + 1))
                pend_g[i + 1] = issue_gathers(i + 1)
            drain(pend_g.pop(i))
            if i >= 1 and (i - 1) in pend_s:
                drain(pend_s.pop(i - 1))
            if i + 1 < n_chunks:
                pend_dst[i + 1] = issue_dst(i + 1)
            drain(pend_dst.pop(i))
            compute(i)
            pend_s[i] = issue_scatters(i)
            if i + 2 < n_chunks:
                pend_sp[i + 2] = issue_sp(i + 2)
        for i in sorted(pend_s):
            drain(pend_s[i])

        plsc.subcore_barrier()
        r0 = sid * n_read
        pltpu.sync_copy(agg_sh.at[pl.ds(r0, n_read)],
                        agg_out.at[pl.ds(cid * n_nodes + r0, n_read)])

    return edge_kernel


def kernel(x, edge_index, edge_attr, W1, root1, b1, W2, root2, b2):
    n, f_in = x.shape
    e = edge_index.shape[1]
    hid = W1.shape[2]
    ncls = W2.shape[2]
    bn = 1000

    n_pad = ((n + 127) // 128) * 128
    rpt = ((e + SUB * NWORKERS * CHUNK_ROWS - 1) // (SUB * NWORKERS * CHUNK_ROWS)) * CHUNK_ROWS
    e_pad = rpt * NWORKERS * SUB
    padn = e_pad - e
    src2 = jnp.concatenate([edge_index[0], jnp.zeros((padn,), jnp.int32)]).reshape(e_pad // SUB, SUB)
    dst2 = jnp.concatenate([edge_index[1], jnp.full((padn,), n, jnp.int32)]).reshape(e_pad // SUB, SUB)
    p = jnp.concatenate([edge_attr[:, 0], jnp.zeros((padn,), jnp.float32)])

    wud1 = jnp.concatenate([W1[0], W1[1] - W1[0]], axis=1)
    wud2 = jnp.concatenate([W2[0], W2[1] - W2[0]], axis=1)
    zf1 = jnp.zeros((n_pad, hid + 16), jnp.float32)
    zf2 = jnp.zeros((n_pad, ncls), jnp.float32)
    c_edges = CHUNK_ROWS * SUB
    pat = jnp.zeros((c_edges, hid + 16), jnp.float32).at[:, hid].set(1.0)

    t1, r1 = _tc_proj(x, wud1, root1, b1.reshape(1, hid), bn)

    edge1 = _make_edge_kernel(n_pad, e_pad // SUB, hid, with_cnt=True)
    aggp1 = edge1(t1, src2, dst2, p, zf1, pat).reshape(2, n_pad, hid + 16)

    t2, r2 = _tc_mid(aggp1, r1, wud2, root2, b2.reshape(1, ncls), bn)

    edge2 = _make_edge_kernel(n_pad, e_pad // SUB, ncls, with_cnt=False)
    aggp2 = edge2(t2, src2, dst2, p, zf2)
    aggp2 = aggp2.reshape(2, n_pad, ncls)

    cntp = aggp1[:, :, hid:hid + 1]
    return _tc_out(aggp2, cntp, r2, bn)
